# baseline (speedup 1.0000x reference)
import functools

import jax
import jax.numpy as jnp
from jax import lax
from jax.experimental import pallas as pl
from jax.experimental.pallas import tpu as pltpu
from jax.experimental.pallas import tpu_sc as plsc

N = 4096
M = 4096

NC = 2
NS = 16
L = 16
NW = NC * NS
RSC = 1536
RPW = RSC // NW
CHUNK = 8
NBUF = 3
NCHUNK = RPW // CHUNK
KV = N // L

_mesh = plsc.VectorSubcoreMesh(
    core_axis_name="c", subcore_axis_name="s", num_cores=NC, num_subcores=NS)


@functools.partial(
    pl.kernel,
    out_type=jax.ShapeDtypeStruct((NW, 2, L), jnp.float32),
    mesh=_mesh,
    scratch_types=[
        pltpu.VMEM((N,), jnp.float32),
        pltpu.VMEM((RPW,), jnp.float32),
        pltpu.VMEM((RPW,), jnp.float32),
        pltpu.VMEM((NBUF, CHUNK, N), jnp.float32),
        pltpu.VMEM((L, L), jnp.float32),
        pltpu.VMEM((2, L), jnp.float32),
        pltpu.SemaphoreType.DMA,
        pltpu.SemaphoreType.DMA,
        pltpu.SemaphoreType.DMA,
    ],
    compiler_params=pltpu.CompilerParams(needs_layout_passes=False),
)
def _rel_kkt_sc(a_hbm, x_hbm, b_hbm, iy_hbm, out_hbm,
                x_v, b_v, iy_v, abuf, dots_v, st_v, sem0, sem1, sem2):
    wid = lax.axis_index("s") * NC + lax.axis_index("c")
    row0 = wid * RPW
    sems = (sem0, sem1, sem2)

    pltpu.sync_copy(x_hbm, x_v)
    pltpu.sync_copy(b_hbm.at[pl.ds(row0, RPW)], b_v)
    pltpu.sync_copy(iy_hbm.at[pl.ds(row0, RPW)], iy_v)

    def start(g):
        return pltpu.async_copy(
            a_hbm.at[pl.ds(row0 + g * CHUNK, CHUNK)],
            abuf.at[g % NBUF], sems[g % NBUF])

    handles = {}
    for g in range(min(NBUF, NCHUNK)):
        handles[g] = start(g)

    zero = jnp.zeros((L,), jnp.float32)
    lanes = lax.iota(jnp.int32, L)
    totacc = zero
    for g in range(NCHUNK):
        handles.pop(g).wait()
        buf = abuf.at[g % NBUF]

        def body(k, accs, buf=buf):
            xk = x_v[pl.ds(k * L, L)]
            return tuple(accs[u] + buf[u, pl.ds(k * L, L)] * xk
                         for u in range(CHUNK))

        accs = lax.fori_loop(
            0, KV, body, tuple(zero for _ in range(CHUNK)))

        base = (g % 2) * CHUNK
        for u in range(CHUNK):
            dots_v[base + u, :] = accs[u]

        if g % 2 == 1:
            rowsums = zero
            for cc in range(L):
                col = jnp.full((L,), cc, jnp.int32)
                rowsums = rowsums + plsc.load_gather(dots_v, [lanes, col])
            r0 = (g - 1) * CHUNK
            bq = b_v[pl.ds(r0, L)]
            iq = iy_v[pl.ds(r0, L)]
            v = rowsums - bq
            f = v + iq * jnp.maximum(-v, zero)
            totacc = totacc + jnp.abs(f)

        if g + NBUF < NCHUNK:
            handles[g + NBUF] = start(g + NBUF)

    bacc = zero
    for j in range(RPW // L):
        bacc = bacc + jnp.abs(b_v[pl.ds(j * L, L)])

    st_v[0, :] = totacc
    st_v[1, :] = bacc
    pltpu.sync_copy(st_v, out_hbm.at[wid])


BLK = 512
TC_ROWS = M - RSC
TC_GRID = TC_ROWS // BLK
TC_OFF = RSC // BLK


def _tc_body(a_ref, x_ref, b_ref, iy_ref, num_ref, bs_ref):
    i = pl.program_id(0)
    ax = jnp.sum(a_ref[...] * x_ref[...], axis=1, keepdims=True)
    v = ax - b_ref[...]
    f = v + iy_ref[...] * jnp.maximum(-v, 0.0)

    @pl.when(i == 0)
    def _():
        num_ref[...] = jnp.zeros((1, 1), jnp.float32)
        bs_ref[...] = jnp.zeros((1, 1), jnp.float32)

    num_ref[...] += jnp.sum(jnp.abs(f), keepdims=True)
    bs_ref[...] += jnp.sum(jnp.abs(b_ref[...]), keepdims=True)


_tc_call = pl.pallas_call(
    _tc_body,
    grid=(TC_GRID,),
    in_specs=[
        pl.BlockSpec((BLK, N), lambda i: (i + TC_OFF, 0)),
        pl.BlockSpec((1, N), lambda i: (0, 0)),
        pl.BlockSpec((BLK, 1), lambda i: (i + TC_OFF, 0)),
        pl.BlockSpec((BLK, 1), lambda i: (i + TC_OFF, 0)),
    ],
    out_specs=[
        pl.BlockSpec((1, 1), lambda i: (0, 0)),
        pl.BlockSpec((1, 1), lambda i: (0, 0)),
    ],
    out_shape=[
        jax.ShapeDtypeStruct((1, 1), jnp.float32),
        jax.ShapeDtypeStruct((1, 1), jnp.float32),
    ],
)


def kernel(Q, A, AT, b, c, x, y, Iy):
    sc_parts = _rel_kkt_sc(A, x.reshape(-1), b, Iy)
    tc_num, tc_bs = _tc_call(A, x.reshape(1, N), b.reshape(M, 1),
                             Iy.reshape(M, 1))
    num = jnp.sum(sc_parts[:, 0, :]) + tc_num[0, 0]
    bsum = jnp.sum(sc_parts[:, 1, :]) + tc_bs[0, 0]
    return num / (jnp.float32(1.0) + bsum)


# speedup vs baseline: 1.0068x; 1.0068x over previous
import functools

import jax
import jax.numpy as jnp
from jax import lax
from jax.experimental import pallas as pl
from jax.experimental.pallas import tpu as pltpu
from jax.experimental.pallas import tpu_sc as plsc

N = 4096
M = 4096

NC = 2
NS = 16
L = 16
NW = NC * NS
RSC = 1536
RPW = RSC // NW
CHUNK = 8
NBUF = 2
NCHUNK = RPW // CHUNK
KV = N // L

_mesh = plsc.VectorSubcoreMesh(
    core_axis_name="c", subcore_axis_name="s", num_cores=NC, num_subcores=NS)


@functools.partial(
    pl.kernel,
    out_type=jax.ShapeDtypeStruct((NW, 2, L), jnp.float32),
    mesh=_mesh,
    scratch_types=[
        pltpu.VMEM((N,), jnp.float32),
        pltpu.VMEM((RPW,), jnp.float32),
        pltpu.VMEM((RPW,), jnp.float32),
        pltpu.VMEM((NBUF, CHUNK, N), jnp.float32),
        pltpu.VMEM((L, L), jnp.float32),
        pltpu.VMEM((2, L), jnp.float32),
        pltpu.SemaphoreType.DMA,
        pltpu.SemaphoreType.DMA,
    ],
    compiler_params=pltpu.CompilerParams(needs_layout_passes=False),
)
def _rel_kkt_sc(a_hbm, x_hbm, b_hbm, iy_hbm, out_hbm,
                x_v, b_v, iy_v, abuf, dots_v, st_v, sem0, sem1):
    wid = lax.axis_index("s") * NC + lax.axis_index("c")
    row0 = wid * RPW
    sems = (sem0, sem1)

    pltpu.sync_copy(x_hbm, x_v)
    pltpu.sync_copy(b_hbm.at[pl.ds(row0, RPW)], b_v)
    pltpu.sync_copy(iy_hbm.at[pl.ds(row0, RPW)], iy_v)

    def chunk_copy(g, buf_idx):
        return pltpu.make_async_copy(
            a_hbm.at[pl.ds(row0 + g * CHUNK, CHUNK)],
            abuf.at[buf_idx], sems[buf_idx])

    for bi in range(NBUF):
        chunk_copy(bi, bi).start()

    zero = jnp.zeros((L,), jnp.float32)
    lanes = lax.iota(jnp.int32, L)

    @pl.loop(0, NCHUNK, init_carry=zero, step=2)
    def totacc(g, totacc):
        for bi in range(2):
            gg = g + bi
            chunk_copy(gg, bi).wait()
            buf = abuf.at[bi]

            def body(k, accs, buf=buf):
                xk = x_v[pl.ds(k * L, L)]
                return tuple(accs[u] + buf[u, pl.ds(k * L, L)] * xk
                             for u in range(CHUNK))

            accs = lax.fori_loop(
                0, KV, body, tuple(zero for _ in range(CHUNK)))

            for u in range(CHUNK):
                dots_v[bi * CHUNK + u, :] = accs[u]

            @pl.when(gg + NBUF < NCHUNK)
            def _(gg=gg, bi=bi):
                chunk_copy(gg + NBUF, bi).start()

        rowsums = zero
        for cc in range(L):
            col = jnp.full((L,), cc, jnp.int32)
            rowsums = rowsums + plsc.load_gather(dots_v, [lanes, col])
        bq = b_v[pl.ds(g * CHUNK, L)]
        iq = iy_v[pl.ds(g * CHUNK, L)]
        v = rowsums - bq
        f = v + iq * jnp.maximum(-v, zero)
        return totacc + jnp.abs(f)

    bacc = zero
    for j in range(RPW // L):
        bacc = bacc + jnp.abs(b_v[pl.ds(j * L, L)])

    st_v[0, :] = totacc
    st_v[1, :] = bacc
    pltpu.sync_copy(st_v, out_hbm.at[wid])


BLK = 512
TC_ROWS = M - RSC
TC_GRID = TC_ROWS // BLK
TC_OFF = RSC // BLK


def _tc_body(a_ref, x_ref, b_ref, iy_ref, num_ref, bs_ref):
    i = pl.program_id(0)
    ax = jnp.sum(a_ref[...] * x_ref[...], axis=1, keepdims=True)
    v = ax - b_ref[...]
    f = v + iy_ref[...] * jnp.maximum(-v, 0.0)

    @pl.when(i == 0)
    def _():
        num_ref[...] = jnp.zeros((1, 1), jnp.float32)
        bs_ref[...] = jnp.zeros((1, 1), jnp.float32)

    num_ref[...] += jnp.sum(jnp.abs(f), keepdims=True)
    bs_ref[...] += jnp.sum(jnp.abs(b_ref[...]), keepdims=True)


_tc_call = pl.pallas_call(
    _tc_body,
    grid=(TC_GRID,),
    in_specs=[
        pl.BlockSpec((BLK, N), lambda i: (i + TC_OFF, 0)),
        pl.BlockSpec((1, N), lambda i: (0, 0)),
        pl.BlockSpec((BLK, 1), lambda i: (i + TC_OFF, 0)),
        pl.BlockSpec((BLK, 1), lambda i: (i + TC_OFF, 0)),
    ],
    out_specs=[
        pl.BlockSpec((1, 1), lambda i: (0, 0)),
        pl.BlockSpec((1, 1), lambda i: (0, 0)),
    ],
    out_shape=[
        jax.ShapeDtypeStruct((1, 1), jnp.float32),
        jax.ShapeDtypeStruct((1, 1), jnp.float32),
    ],
)


def kernel(Q, A, AT, b, c, x, y, Iy):
    sc_parts = _rel_kkt_sc(A, x.reshape(-1), b, Iy)
    tc_num, tc_bs = _tc_call(A, x.reshape(1, N), b.reshape(M, 1),
                             Iy.reshape(M, 1))
    num = jnp.sum(sc_parts[:, 0, :]) + tc_num[0, 0]
    bsum = jnp.sum(sc_parts[:, 1, :]) + tc_bs[0, 0]
    return num / (jnp.float32(1.0) + bsum)


# speedup vs baseline: 2.1813x; 2.1665x over previous
import jax
import jax.numpy as jnp
from jax.experimental import pallas as pl
from jax.experimental.pallas import tpu as pltpu

N = 4096
M = 4096
RB = 128
NB = 4
NCH = M // RB


def _body(a_hbm, x_hbm, b_hbm, iy_hbm, out_ref,
          abuf, x_v, b_v, iy_v, *sems):
    pltpu.make_async_copy(x_hbm, x_v, sems[NB]).start()
    pltpu.make_async_copy(b_hbm, b_v, sems[NB + 1]).start()
    pltpu.make_async_copy(iy_hbm, iy_v, sems[NB + 2]).start()

    def chunk_copy(g, bi):
        return pltpu.make_async_copy(
            a_hbm.at[pl.ds(g * RB, RB)], abuf.at[bi], sems[bi])

    for bi in range(NB):
        chunk_copy(bi, bi).start()

    pltpu.make_async_copy(x_hbm, x_v, sems[NB]).wait()
    pltpu.make_async_copy(b_hbm, b_v, sems[NB + 1]).wait()
    pltpu.make_async_copy(iy_hbm, iy_v, sems[NB + 2]).wait()
    xr = x_v[...]

    @pl.loop(0, NCH, init_carry=jnp.float32(0.0), step=NB)
    def tot(g, tot):
        for bi in range(NB):
            gg = g + bi
            chunk_copy(gg, bi).wait()
            ax = jnp.sum(abuf[bi] * xr, axis=1)
            v = ax - b_v[0, pl.ds(gg * RB, RB)]
            f = v + iy_v[0, pl.ds(gg * RB, RB)] * jnp.maximum(-v, 0.0)
            tot = tot + jnp.sum(jnp.abs(f))

            @pl.when(gg + NB < NCH)
            def _(gg=gg, bi=bi):
                chunk_copy(gg + NB, bi).start()
        return tot

    bsum = jnp.sum(jnp.abs(b_v[...]))
    out_ref[...] = jnp.full((1, 1), tot / (1.0 + bsum), jnp.float32)


_call = pl.pallas_call(
    _body,
    in_specs=[
        pl.BlockSpec(memory_space=pl.ANY),
        pl.BlockSpec(memory_space=pl.ANY),
        pl.BlockSpec(memory_space=pl.ANY),
        pl.BlockSpec(memory_space=pl.ANY),
    ],
    out_specs=pl.BlockSpec(memory_space=pltpu.MemorySpace.VMEM),
    out_shape=jax.ShapeDtypeStruct((1, 1), jnp.float32),
    scratch_shapes=[
        pltpu.VMEM((NB, RB, N), jnp.float32),
        pltpu.VMEM((1, N), jnp.float32),
        pltpu.VMEM((1, M), jnp.float32),
        pltpu.VMEM((1, M), jnp.float32),
    ] + [pltpu.SemaphoreType.DMA] * (NB + 3),
)


def kernel(Q, A, AT, b, c, x, y, Iy):
    res = _call(A, x.reshape(1, N), b.reshape(1, M), Iy.reshape(1, M))
    return res[0, 0]
